# baseline (speedup 1.0000x reference)
import functools

import jax
import jax.numpy as jnp
from jax import lax
from jax.experimental import pallas as pl
from jax.experimental.pallas import tpu as pltpu
from jax.experimental.pallas import tpu_sc as plsc

NUM_NODES = 100000
EMBED_DIM = 128
NUM_WORKERS = 32
CHUNK_ROWS = 1000
NUM_CHUNKS = NUM_NODES // CHUNK_ROWS
MAX_CHUNKS_PER_WORKER = -(-NUM_CHUNKS // NUM_WORKERS)


def kernel(embedding_table):
    n, d = embedding_table.shape
    mesh = plsc.VectorSubcoreMesh(core_axis_name="c", subcore_axis_name="s")

    @functools.partial(
        pl.kernel,
        mesh=mesh,
        out_type=jax.ShapeDtypeStruct((n, d), embedding_table.dtype),
    )
    def copy_k(table_hbm, out_hbm):
        wid = lax.axis_index("s") * 2 + lax.axis_index("c")
        for k in range(MAX_CHUNKS_PER_WORKER):
            c = wid + k * NUM_WORKERS

            @pl.when(c < NUM_CHUNKS)
            def _():
                base = c * CHUNK_ROWS
                pltpu.sync_copy(
                    table_hbm.at[pl.ds(base, CHUNK_ROWS)],
                    out_hbm.at[pl.ds(base, CHUNK_ROWS)],
                )

    return copy_k(embedding_table)


# speedup vs baseline: 29.0674x; 29.0674x over previous
import functools

import jax
import jax.numpy as jnp
from jax import lax
from jax.experimental import pallas as pl
from jax.experimental.pallas import tpu as pltpu
from jax.experimental.pallas import tpu_sc as plsc

NUM_NODES = 100000
EMBED_DIM = 128
NUM_CORES = 2
NUM_SUBCORES = 16
NUM_WORKERS = NUM_CORES * NUM_SUBCORES
CHUNK_ROWS = 400
NUM_CHUNKS = NUM_NODES // CHUNK_ROWS
MAX_K = -(-NUM_CHUNKS // NUM_WORKERS)
NBUF = 2


def kernel(embedding_table):
    n, d = embedding_table.shape
    mesh = plsc.VectorSubcoreMesh(core_axis_name="c", subcore_axis_name="s")

    @functools.partial(
        pl.kernel,
        mesh=mesh,
        out_type=jax.ShapeDtypeStruct((n, d), embedding_table.dtype),
        scratch_types=[
            pltpu.VMEM((NBUF, CHUNK_ROWS, EMBED_DIM), jnp.float32),
            pltpu.SemaphoreType.DMA((NBUF,)),
            pltpu.SemaphoreType.DMA((NBUF,)),
        ],
    )
    def copy_k(table_hbm, out_hbm, bufs, in_sems, out_sems):
        wid = lax.axis_index("s") * NUM_CORES + lax.axis_index("c")

        def in_dma(k, slot):
            c = wid + k * NUM_WORKERS
            return pltpu.make_async_copy(
                table_hbm.at[pl.ds(c * CHUNK_ROWS, CHUNK_ROWS)],
                bufs.at[slot],
                in_sems.at[slot],
            )

        def out_dma(k, slot):
            c = wid + k * NUM_WORKERS
            return pltpu.make_async_copy(
                bufs.at[slot],
                out_hbm.at[pl.ds(c * CHUNK_ROWS, CHUNK_ROWS)],
                out_sems.at[slot],
            )

        def valid(k):
            return wid + k * NUM_WORKERS < NUM_CHUNKS

        @pl.when(valid(0))
        def _():
            in_dma(0, 0).start()

        for k in range(MAX_K):
            slot = k % NBUF
            nslot = (k + 1) % NBUF
            if k + 1 < MAX_K:
                @pl.when(valid(k + 1))
                def _(k=k, nslot=nslot):
                    if k >= 1:
                        out_dma(k - 1, nslot).wait()
                    in_dma(k + 1, nslot).start()

            @pl.when(valid(k))
            def _(k=k, slot=slot):
                in_dma(k, slot).wait()
                out_dma(k, slot).start()

        for k in (MAX_K - 2, MAX_K - 1):
            @pl.when(valid(k))
            def _(k=k):
                out_dma(k, k % NBUF).wait()

    return copy_k(embedding_table)
